# Initial kernel scaffold; baseline (speedup 1.0000x reference)
import functools

import jax
import jax.numpy as jnp
from jax import lax
from jax.experimental import pallas as pl
from jax.experimental.pallas import tpu as pltpu
from jax.experimental.pallas import tpu_sc as plsc

N = 10000
E = 320000
D = 128
NC = 2
NS = 16
NW = NC * NS
LANES = 16
EDGES_PER_TILE = E // NW
CHUNK = 80
NCHUNK = EDGES_PER_TILE // CHUNK
ROWS_PER_TILE = N // NS


def _lane_broadcast(v16, e):
    idx = jnp.full((LANES, 1), e, jnp.int32)
    dn = lax.GatherDimensionNumbers(
        offset_dims=(), collapsed_slice_dims=(0,), start_index_map=(0,))
    return lax.gather(v16, idx, dn, (1,),
                      mode=lax.GatherScatterMode.PROMISE_IN_BOUNDS)


def _spmm_partials(emb, src, dst, vals):
    mesh = plsc.VectorSubcoreMesh(
        core_axis_name="c", subcore_axis_name="s", num_cores=NC,
        num_subcores=NS)

    @functools.partial(
        pl.kernel,
        out_type=jax.ShapeDtypeStruct((NC * N, D), jnp.float32),
        mesh=mesh,
        scratch_types=[
            pltpu.VMEM_SHARED((N, D), jnp.float32),
            pltpu.VMEM((CHUNK,), jnp.int32),
            pltpu.VMEM((CHUNK,), jnp.int32),
            pltpu.VMEM((CHUNK,), jnp.float32),
            pltpu.VMEM((CHUNK, D), jnp.float32),
            pltpu.VMEM((ROWS_PER_TILE, D), jnp.float32),
            pltpu.SemaphoreType.DMA,
        ],
    )
    def spmm(emb_hbm, src_hbm, dst_hbm, val_hbm, out_hbm,
             acc_sh, src_v, dst_v, val_v, rows_v, zero_v, sem):
        c = lax.axis_index("c")
        s = lax.axis_index("s")
        wid = s * NC + c

        zeros16 = jnp.zeros((LANES,), jnp.float32)

        def zrow(r, carry):
            for j in range(D // LANES):
                zero_v[r, pl.ds(j * LANES, LANES)] = zeros16
            return carry

        lax.fori_loop(0, ROWS_PER_TILE, zrow, 0)
        pltpu.sync_copy(zero_v,
                        acc_sh.at[pl.ds(s * ROWS_PER_TILE, ROWS_PER_TILE)])
        plsc.subcore_barrier()

        base = wid * EDGES_PER_TILE

        def chunk_body(i, carry):
            off = base + i * CHUNK
            pltpu.sync_copy(src_hbm.at[pl.ds(off, CHUNK)], src_v)
            pltpu.sync_copy(dst_hbm.at[pl.ds(off, CHUNK)], dst_v)
            pltpu.sync_copy(val_hbm.at[pl.ds(off, CHUNK)], val_v)
            pltpu.async_copy(emb_hbm.at[src_v], rows_v, sem).wait()

            def group(g, gcarry):
                a16 = val_v[pl.ds(g * LANES, LANES)]
                for e in range(LANES):
                    ae = _lane_broadcast(a16, e)
                    r = g * LANES + e
                    for j in range(D // LANES):
                        sl = pl.ds(j * LANES, LANES)
                        rows_v[r, sl] = rows_v[r, sl] * ae
                return gcarry

            lax.fori_loop(0, CHUNK // LANES, group, 0)
            pltpu.sync_copy(rows_v, acc_sh.at[dst_v], add=True)
            return carry

        lax.fori_loop(0, NCHUNK, chunk_body, 0)
        plsc.subcore_barrier()

        pltpu.sync_copy(
            acc_sh.at[pl.ds(s * ROWS_PER_TILE, ROWS_PER_TILE)],
            out_hbm.at[pl.ds(c * N + s * ROWS_PER_TILE, ROWS_PER_TILE)])

    return spmm(emb, src, dst, vals)


_BLK = 1000


def _dense_body(pa_ref, pb_ref, w_ref, o_ref):
    x = pa_ref[...] + pb_ref[...]
    y = jnp.dot(x, w_ref[...], preferred_element_type=jnp.float32)
    o_ref[...] = 1.0 / (1.0 + jnp.exp(-y))


def _dense(partials, W):
    nblk = N // _BLK
    return pl.pallas_call(
        _dense_body,
        grid=(nblk,),
        in_specs=[
            pl.BlockSpec((_BLK, D), lambda i: (i, 0)),
            pl.BlockSpec((_BLK, D), lambda i, nblk=nblk: (i + nblk, 0)),
            pl.BlockSpec((D, D), lambda i: (0, 0)),
        ],
        out_specs=pl.BlockSpec((_BLK, D), lambda i: (i, 0)),
        out_shape=jax.ShapeDtypeStruct((N, D), jnp.float32),
    )(partials, partials, W)


def _dense_final_body(pa_ref, pb_ref, w_ref, x0_ref, x1_ref, x2_ref,
                      o3_ref, om_ref):
    x = pa_ref[...] + pb_ref[...]
    y = jnp.dot(x, w_ref[...], preferred_element_type=jnp.float32)
    e3 = 1.0 / (1.0 + jnp.exp(-y))
    o3_ref[...] = e3
    om_ref[...] = (x0_ref[...] + x1_ref[...] + x2_ref[...] + e3) * 0.25


def _dense_final(partials, W, e0, e1, e2):
    nblk = N // _BLK
    row_spec = pl.BlockSpec((_BLK, D), lambda i: (i, 0))
    return pl.pallas_call(
        _dense_final_body,
        grid=(nblk,),
        in_specs=[
            row_spec,
            pl.BlockSpec((_BLK, D), lambda i, nblk=nblk: (i + nblk, 0)),
            pl.BlockSpec((D, D), lambda i: (0, 0)),
            row_spec, row_spec, row_spec,
        ],
        out_specs=[row_spec, row_spec],
        out_shape=[jax.ShapeDtypeStruct((N, D), jnp.float32),
                   jax.ShapeDtypeStruct((N, D), jnp.float32)],
    )(partials, partials, W, e0, e1, e2)


def kernel(ItemAndUserEmebddings, edge_index, A_values, W0, W1, W2):
    x = ItemAndUserEmebddings
    src = edge_index[0]
    dst = edge_index[1]

    p = _spmm_partials(x, src, dst, A_values)
    e1 = _dense(p, W0)
    p = _spmm_partials(e1, src, dst, A_values)
    e2 = _dense(p, W1)
    p = _spmm_partials(e2, src, dst, A_values)
    e3, mean = _dense_final(p, W2, x, e1, e2)
    return (mean, x, e1, e2, e3)


# speedup vs baseline: 4.1414x; 4.1414x over previous
import functools

import jax
import jax.numpy as jnp
from jax import lax
from jax.experimental import pallas as pl
from jax.experimental.pallas import tpu as pltpu
from jax.experimental.pallas import tpu_sc as plsc

N = 10000
E = 320000
D = 128
NC = 2
NS = 16
NW = NC * NS
LANES = 16
EDGES_PER_TILE = E // NW
CHUNK = 80
NCHUNK = EDGES_PER_TILE // CHUNK
NPAD = 10240
ROWS_PER_TILE = NPAD // NS


def _lane_broadcast(v16, e):
    idx = jnp.full((LANES, 1), e, jnp.int32)
    dn = lax.GatherDimensionNumbers(
        offset_dims=(), collapsed_slice_dims=(0,), start_index_map=(0,))
    return lax.gather(v16, idx, dn, (1,),
                      mode=lax.GatherScatterMode.PROMISE_IN_BOUNDS)


def _spmm_partials(emb, src, dst, vals):
    mesh = plsc.VectorSubcoreMesh(
        core_axis_name="c", subcore_axis_name="s", num_cores=NC,
        num_subcores=NS)

    @functools.partial(
        pl.kernel,
        out_type=jax.ShapeDtypeStruct((NC, NPAD, D), jnp.float32),
        mesh=mesh,
        compiler_params=pltpu.CompilerParams(use_tc_tiling_on_sc=False),
        scratch_types=[
            pltpu.VMEM_SHARED((NPAD, D), jnp.float32),
            pltpu.VMEM((CHUNK,), jnp.int32),
            pltpu.VMEM((CHUNK,), jnp.int32),
            pltpu.VMEM((CHUNK,), jnp.float32),
            pltpu.VMEM((CHUNK, D), jnp.float32),
            pltpu.SemaphoreType.DMA,
        ],
    )
    def spmm(emb_hbm, src_hbm, dst_hbm, val_hbm, out_hbm,
             acc_sh, src_v, dst_v, val_v, rows_v, sem):
        c = lax.axis_index("c")
        s = lax.axis_index("s")
        wid = s * NC + c

        zeros16 = jnp.zeros((LANES,), jnp.float32)

        def zrow(r, carry):
            for j in range(D // LANES):
                rows_v[r, pl.ds(j * LANES, LANES)] = zeros16
            return carry

        lax.fori_loop(0, CHUNK, zrow, 0)
        for t in range(ROWS_PER_TILE // CHUNK):
            pltpu.sync_copy(
                rows_v, acc_sh.at[pl.ds(s * ROWS_PER_TILE + t * CHUNK, CHUNK)])
        plsc.subcore_barrier()

        base = wid * EDGES_PER_TILE

        def chunk_body(i, carry):
            off = base + i * CHUNK
            pltpu.sync_copy(src_hbm.at[pl.ds(off, CHUNK)], src_v)
            pltpu.sync_copy(dst_hbm.at[pl.ds(off, CHUNK)], dst_v)
            pltpu.sync_copy(val_hbm.at[pl.ds(off, CHUNK)], val_v)
            pltpu.async_copy(emb_hbm.at[src_v], rows_v, sem).wait()

            def group(g, gcarry):
                a16 = val_v[pl.ds(g * LANES, LANES)]
                for e in range(LANES):
                    ae = _lane_broadcast(a16, e)
                    r = g * LANES + e
                    for j in range(D // LANES):
                        sl = pl.ds(j * LANES, LANES)
                        rows_v[r, sl] = rows_v[r, sl] * ae
                return gcarry

            lax.fori_loop(0, CHUNK // LANES, group, 0)
            pltpu.sync_copy(rows_v, acc_sh.at[dst_v], add=True)
            return carry

        lax.fori_loop(0, NCHUNK, chunk_body, 0)
        plsc.subcore_barrier()

        pltpu.sync_copy(
            acc_sh.at[pl.ds(s * ROWS_PER_TILE, ROWS_PER_TILE)],
            out_hbm.at[c, pl.ds(s * ROWS_PER_TILE, ROWS_PER_TILE)])

    return spmm(emb, src, dst, vals)


_BLK = 1000


_P_SPEC_A = pl.BlockSpec((1, _BLK, D), lambda i: (0, i, 0))
_P_SPEC_B = pl.BlockSpec((1, _BLK, D), lambda i: (1, i, 0))
_W_SPEC = pl.BlockSpec((D, D), lambda i: (0, 0))
_ROW_SPEC = pl.BlockSpec((_BLK, D), lambda i: (i, 0))


def _dense_body(pa_ref, pb_ref, w_ref, o_ref):
    x = pa_ref[0] + pb_ref[0]
    y = jnp.dot(x, w_ref[...], preferred_element_type=jnp.float32)
    o_ref[...] = 1.0 / (1.0 + jnp.exp(-y))


def _dense(partials, W):
    return pl.pallas_call(
        _dense_body,
        grid=(N // _BLK,),
        in_specs=[_P_SPEC_A, _P_SPEC_B, _W_SPEC],
        out_specs=_ROW_SPEC,
        out_shape=jax.ShapeDtypeStruct((N, D), jnp.float32),
    )(partials, partials, W)


def _dense_final_body(pa_ref, pb_ref, w_ref, x0_ref, x1_ref, x2_ref,
                      o3_ref, om_ref):
    x = pa_ref[0] + pb_ref[0]
    y = jnp.dot(x, w_ref[...], preferred_element_type=jnp.float32)
    e3 = 1.0 / (1.0 + jnp.exp(-y))
    o3_ref[...] = e3
    om_ref[...] = (x0_ref[...] + x1_ref[...] + x2_ref[...] + e3) * 0.25


def _dense_final(partials, W, e0, e1, e2):
    return pl.pallas_call(
        _dense_final_body,
        grid=(N // _BLK,),
        in_specs=[_P_SPEC_A, _P_SPEC_B, _W_SPEC,
                  _ROW_SPEC, _ROW_SPEC, _ROW_SPEC],
        out_specs=[_ROW_SPEC, _ROW_SPEC],
        out_shape=[jax.ShapeDtypeStruct((N, D), jnp.float32),
                   jax.ShapeDtypeStruct((N, D), jnp.float32)],
    )(partials, partials, W, e0, e1, e2)


def kernel(ItemAndUserEmebddings, edge_index, A_values, W0, W1, W2):
    x = ItemAndUserEmebddings
    src = edge_index[0]
    dst = edge_index[1]

    p = _spmm_partials(x, src, dst, A_values)
    e1 = _dense(p, W0)
    p = _spmm_partials(e1, src, dst, A_values)
    e2 = _dense(p, W1)
    p = _spmm_partials(e2, src, dst, A_values)
    e3, mean = _dense_final(p, W2, x, e1, e2)
    return (mean, x, e1, e2, e3)


# speedup vs baseline: 6.4437x; 1.5559x over previous
import functools

import jax
import jax.numpy as jnp
from jax import lax
from jax.experimental import pallas as pl
from jax.experimental.pallas import tpu as pltpu
from jax.experimental.pallas import tpu_sc as plsc

N = 10000
E = 320000
D = 128
NC = 2
NS = 16
NW = NC * NS
LANES = 16
EDGES_PER_TILE = E // NW
CHUNK = 80
NCHUNK = EDGES_PER_TILE // CHUNK
NPAD = 10240
ROWS_PER_TILE = NPAD // NS


def _lane_broadcast(v16, e):
    idx = jnp.full((LANES, 1), e, jnp.int32)
    dn = lax.GatherDimensionNumbers(
        offset_dims=(), collapsed_slice_dims=(0,), start_index_map=(0,))
    return lax.gather(v16, idx, dn, (1,),
                      mode=lax.GatherScatterMode.PROMISE_IN_BOUNDS)


def _spmm_partials(emb, src, dst, vals):
    mesh = plsc.VectorSubcoreMesh(
        core_axis_name="c", subcore_axis_name="s", num_cores=NC,
        num_subcores=NS)

    @functools.partial(
        pl.kernel,
        out_type=jax.ShapeDtypeStruct((NC, NPAD, D), jnp.float32),
        mesh=mesh,
        compiler_params=pltpu.CompilerParams(use_tc_tiling_on_sc=False),
        scratch_types=[
            pltpu.VMEM_SHARED((NPAD, D), jnp.float32),
            pltpu.VMEM((NCHUNK, CHUNK), jnp.int32),
            pltpu.VMEM((NCHUNK, CHUNK), jnp.int32),
            pltpu.VMEM((EDGES_PER_TILE,), jnp.float32),
            pltpu.VMEM((CHUNK, D), jnp.float32),
            pltpu.SemaphoreType.DMA,
        ],
    )
    def spmm(emb_hbm, src_hbm, dst_hbm, val_hbm, out_hbm,
             acc_sh, src_v, dst_v, val_v, rows_v, sem):
        c = lax.axis_index("c")
        s = lax.axis_index("s")
        wid = s * NC + c

        pltpu.sync_copy(src_hbm.at[wid], src_v)
        pltpu.sync_copy(dst_hbm.at[wid], dst_v)
        pltpu.sync_copy(val_hbm.at[wid], val_v)

        zeros16 = jnp.zeros((LANES,), jnp.float32)

        def zrow(r, carry):
            for j in range(D // LANES):
                rows_v[r, pl.ds(j * LANES, LANES)] = zeros16
            return carry

        lax.fori_loop(0, CHUNK, zrow, 0)
        for t in range(ROWS_PER_TILE // CHUNK):
            pltpu.sync_copy(
                rows_v, acc_sh.at[pl.ds(s * ROWS_PER_TILE + t * CHUNK, CHUNK)])
        plsc.subcore_barrier()

        def chunk_body(i, carry):
            pltpu.async_copy(emb_hbm.at[src_v.at[i]], rows_v, sem).wait()

            def group(g, gcarry):
                a16 = val_v[pl.ds(i * CHUNK + g * LANES, LANES)]
                for e in range(LANES):
                    ae = _lane_broadcast(a16, e)
                    r = g * LANES + e
                    for j in range(D // LANES):
                        sl = pl.ds(j * LANES, LANES)
                        rows_v[r, sl] = rows_v[r, sl] * ae
                return gcarry

            lax.fori_loop(0, CHUNK // LANES, group, 0)
            pltpu.sync_copy(rows_v, acc_sh.at[dst_v.at[i]], add=True)
            return carry

        lax.fori_loop(0, NCHUNK, chunk_body, 0)
        plsc.subcore_barrier()

        pltpu.sync_copy(
            acc_sh.at[pl.ds(s * ROWS_PER_TILE, ROWS_PER_TILE)],
            out_hbm.at[c, pl.ds(s * ROWS_PER_TILE, ROWS_PER_TILE)])

    return spmm(emb, src, dst, vals)


_BLK = 1000


_P_SPEC_A = pl.BlockSpec((1, _BLK, D), lambda i: (0, i, 0))
_P_SPEC_B = pl.BlockSpec((1, _BLK, D), lambda i: (1, i, 0))
_W_SPEC = pl.BlockSpec((D, D), lambda i: (0, 0))
_ROW_SPEC = pl.BlockSpec((_BLK, D), lambda i: (i, 0))


def _dense_body(pa_ref, pb_ref, w_ref, o_ref):
    x = pa_ref[0] + pb_ref[0]
    y = jnp.dot(x, w_ref[...], preferred_element_type=jnp.float32)
    o_ref[...] = 1.0 / (1.0 + jnp.exp(-y))


def _dense(partials, W):
    return pl.pallas_call(
        _dense_body,
        grid=(N // _BLK,),
        in_specs=[_P_SPEC_A, _P_SPEC_B, _W_SPEC],
        out_specs=_ROW_SPEC,
        out_shape=jax.ShapeDtypeStruct((N, D), jnp.float32),
    )(partials, partials, W)


def _dense_final_body(pa_ref, pb_ref, w_ref, x0_ref, x1_ref, x2_ref,
                      o3_ref, om_ref):
    x = pa_ref[0] + pb_ref[0]
    y = jnp.dot(x, w_ref[...], preferred_element_type=jnp.float32)
    e3 = 1.0 / (1.0 + jnp.exp(-y))
    o3_ref[...] = e3
    om_ref[...] = (x0_ref[...] + x1_ref[...] + x2_ref[...] + e3) * 0.25


def _dense_final(partials, W, e0, e1, e2):
    return pl.pallas_call(
        _dense_final_body,
        grid=(N // _BLK,),
        in_specs=[_P_SPEC_A, _P_SPEC_B, _W_SPEC,
                  _ROW_SPEC, _ROW_SPEC, _ROW_SPEC],
        out_specs=[_ROW_SPEC, _ROW_SPEC],
        out_shape=[jax.ShapeDtypeStruct((N, D), jnp.float32),
                   jax.ShapeDtypeStruct((N, D), jnp.float32)],
    )(partials, partials, W, e0, e1, e2)


def kernel(ItemAndUserEmebddings, edge_index, A_values, W0, W1, W2):
    x = ItemAndUserEmebddings
    src = edge_index[0].reshape(NW, NCHUNK, CHUNK)
    dst = edge_index[1].reshape(NW, NCHUNK, CHUNK)
    A_values = A_values.reshape(NW, EDGES_PER_TILE)

    p = _spmm_partials(x, src, dst, A_values)
    e1 = _dense(p, W0)
    p = _spmm_partials(e1, src, dst, A_values)
    e2 = _dense(p, W1)
    p = _spmm_partials(e2, src, dst, A_values)
    e3, mean = _dense_final(p, W2, x, e1, e2)
    return (mean, x, e1, e2, e3)


# speedup vs baseline: 10.9548x; 1.7001x over previous
import functools

import jax
import jax.numpy as jnp
from jax import lax
from jax.experimental import pallas as pl
from jax.experimental.pallas import tpu as pltpu
from jax.experimental.pallas import tpu_sc as plsc

N = 10000
E = 320000
D = 128
NC = 2
NS = 16
NW = NC * NS
LANES = 16
EDGES_PER_TILE = E // NW
CHUNK = 80
NCHUNK = EDGES_PER_TILE // CHUNK
ROWS_PER_TILE = N // NS


def _lane_broadcast(v16, e):
    idx = jnp.full((LANES, 1), e, jnp.int32)
    dn = lax.GatherDimensionNumbers(
        offset_dims=(), collapsed_slice_dims=(0,), start_index_map=(0,))
    return lax.gather(v16, idx, dn, (1,),
                      mode=lax.GatherScatterMode.PROMISE_IN_BOUNDS)


def _spmm_partials(emb, src, dst, vals):
    mesh = plsc.VectorSubcoreMesh(
        core_axis_name="c", subcore_axis_name="s", num_cores=NC,
        num_subcores=NS)

    @functools.partial(
        pl.kernel,
        out_type=jax.ShapeDtypeStruct((NC, N, D), jnp.float32),
        mesh=mesh,
        compiler_params=pltpu.CompilerParams(use_tc_tiling_on_sc=False),
        scratch_types=[
            pltpu.VMEM_SHARED((N, D), jnp.float32),
            pltpu.VMEM((NCHUNK, CHUNK), jnp.int32),
            pltpu.VMEM((NCHUNK, CHUNK), jnp.int32),
            pltpu.VMEM((EDGES_PER_TILE,), jnp.float32),
            pltpu.VMEM((CHUNK, D), jnp.float32),
            pltpu.VMEM((CHUNK, D), jnp.float32),
            pltpu.SemaphoreType.DMA,
            pltpu.SemaphoreType.DMA,
        ],
    )
    def spmm(emb_hbm, src_hbm, dst_hbm, val_hbm, out_hbm,
             acc_sh, src_v, dst_v, val_v, r0, r1, sem0, sem1):
        c = lax.axis_index("c")
        s = lax.axis_index("s")
        wid = s * NC + c

        pltpu.sync_copy(src_hbm.at[wid], src_v)
        pltpu.sync_copy(dst_hbm.at[wid], dst_v)
        pltpu.sync_copy(val_hbm.at[wid], val_v)

        zeros16 = jnp.zeros((LANES,), jnp.float32)

        def zrow(r, carry):
            for j in range(D // LANES):
                r0[r, pl.ds(j * LANES, LANES)] = zeros16
            return carry

        lax.fori_loop(0, CHUNK, zrow, 0)
        base = s * ROWS_PER_TILE
        for t in range(ROWS_PER_TILE // CHUNK):
            pltpu.sync_copy(r0, acc_sh.at[pl.ds(base + t * CHUNK, CHUNK)])
        rem = ROWS_PER_TILE % CHUNK
        if rem:
            pltpu.sync_copy(
                r0.at[pl.ds(0, rem)],
                acc_sh.at[pl.ds(base + ROWS_PER_TILE - rem, rem)])
        plsc.subcore_barrier()

        def start_gather(ci, rbuf, sem):
            pltpu.async_copy(emb_hbm.at[src_v.at[ci]], rbuf, sem)

        def wait_gather(rbuf, sem):
            pltpu.make_async_copy(emb_hbm.at[src_v.at[0]], rbuf, sem).wait()

        def scale(rbuf, ci):
            def group(g, gcarry):
                a16 = val_v[pl.ds(ci * CHUNK + g * LANES, LANES)]
                for e in range(LANES):
                    ae = _lane_broadcast(a16, e)
                    r = g * LANES + e
                    for j in range(D // LANES):
                        sl = pl.ds(j * LANES, LANES)
                        rbuf[r, sl] = rbuf[r, sl] * ae
                return gcarry

            lax.fori_loop(0, CHUNK // LANES, group, 0)

        def scatter(rbuf, ci):
            pltpu.sync_copy(rbuf, acc_sh.at[dst_v.at[ci]], add=True)

        start_gather(0, r0, sem0)

        def chunk_pair(t, carry):
            i0 = 2 * t
            start_gather(i0 + 1, r1, sem1)
            wait_gather(r0, sem0)
            scale(r0, i0)
            scatter(r0, i0)
            start_gather(i0 + 2, r0, sem0)
            wait_gather(r1, sem1)
            scale(r1, i0 + 1)
            scatter(r1, i0 + 1)
            return carry

        lax.fori_loop(0, (NCHUNK - 1) // 2, chunk_pair, 0)
        wait_gather(r0, sem0)
        scale(r0, NCHUNK - 1)
        scatter(r0, NCHUNK - 1)
        plsc.subcore_barrier()

        pltpu.sync_copy(
            acc_sh.at[pl.ds(s * ROWS_PER_TILE, ROWS_PER_TILE)],
            out_hbm.at[c, pl.ds(s * ROWS_PER_TILE, ROWS_PER_TILE)])

    return spmm(emb, src, dst, vals)


_BLK = 1000


_P_SPEC_A = pl.BlockSpec((1, _BLK, D), lambda i: (0, i, 0))
_P_SPEC_B = pl.BlockSpec((1, _BLK, D), lambda i: (1, i, 0))
_W_SPEC = pl.BlockSpec((D, D), lambda i: (0, 0))
_ROW_SPEC = pl.BlockSpec((_BLK, D), lambda i: (i, 0))


def _dense_body(pa_ref, pb_ref, w_ref, o_ref):
    x = pa_ref[0] + pb_ref[0]
    y = jnp.dot(x, w_ref[...], preferred_element_type=jnp.float32)
    o_ref[...] = 1.0 / (1.0 + jnp.exp(-y))


def _dense(partials, W):
    return pl.pallas_call(
        _dense_body,
        grid=(N // _BLK,),
        in_specs=[_P_SPEC_A, _P_SPEC_B, _W_SPEC],
        out_specs=_ROW_SPEC,
        out_shape=jax.ShapeDtypeStruct((N, D), jnp.float32),
    )(partials, partials, W)


def _dense_final_body(pa_ref, pb_ref, w_ref, x0_ref, x1_ref, x2_ref,
                      o3_ref, om_ref):
    x = pa_ref[0] + pb_ref[0]
    y = jnp.dot(x, w_ref[...], preferred_element_type=jnp.float32)
    e3 = 1.0 / (1.0 + jnp.exp(-y))
    o3_ref[...] = e3
    om_ref[...] = (x0_ref[...] + x1_ref[...] + x2_ref[...] + e3) * 0.25


def _dense_final(partials, W, e0, e1, e2):
    return pl.pallas_call(
        _dense_final_body,
        grid=(N // _BLK,),
        in_specs=[_P_SPEC_A, _P_SPEC_B, _W_SPEC,
                  _ROW_SPEC, _ROW_SPEC, _ROW_SPEC],
        out_specs=[_ROW_SPEC, _ROW_SPEC],
        out_shape=[jax.ShapeDtypeStruct((N, D), jnp.float32),
                   jax.ShapeDtypeStruct((N, D), jnp.float32)],
    )(partials, partials, W, e0, e1, e2)


def kernel(ItemAndUserEmebddings, edge_index, A_values, W0, W1, W2):
    x = ItemAndUserEmebddings
    src = edge_index[0].reshape(NW, NCHUNK, CHUNK)
    dst = edge_index[1].reshape(NW, NCHUNK, CHUNK)
    A_values = A_values.reshape(NW, EDGES_PER_TILE)

    p = _spmm_partials(x, src, dst, A_values)
    e1 = _dense(p, W0)
    p = _spmm_partials(e1, src, dst, A_values)
    e2 = _dense(p, W1)
    p = _spmm_partials(e2, src, dst, A_values)
    e3, mean = _dense_final(p, W2, x, e1, e2)
    return (mean, x, e1, e2, e3)
